# Initial kernel scaffold; baseline (speedup 1.0000x reference)
import functools

import jax
import jax.numpy as jnp
from jax import lax
from jax.experimental import pallas as pl
from jax.experimental.pallas import tpu as pltpu
from jax.experimental.pallas import tpu_sc as plsc

N = 10000
E = 160000
NS = 16
ES = E // NS
F32 = jnp.float32


def _make_prop(D, rows_b, k):
    assert N % rows_b == 0 and rows_b % 16 == 0 and D % 16 == 0 and k % 16 == 0
    nblocks = N // rows_b
    npsc = (nblocks + 1) // 2
    rpw = rows_b // NS
    nscan = ES // 16
    lists_len = ES + 2 * k + 32

    mesh = plsc.VectorSubcoreMesh(core_axis_name="c", subcore_axis_name="s")

    @functools.partial(
        pl.kernel,
        out_type=jax.ShapeDtypeStruct((N, D), F32),
        mesh=mesh,
        scratch_types=[
            pltpu.VMEM((ES,), jnp.int32),
            pltpu.VMEM((ES,), jnp.int32),
            pltpu.VMEM((lists_len,), jnp.int32),
            pltpu.VMEM((lists_len,), jnp.int32),
            pltpu.VMEM((k, D), F32),
            pltpu.VMEM((k,), jnp.int32),
            pltpu.VMEM_SHARED((rows_b + 8, D), F32),
            pltpu.SemaphoreType.DMA,
        ],
    )
    def prop(g_hbm, src_hbm, dst_hbm, out_hbm,
             esrc, edst, srcl, dstl, gbuf, stag, acc, gsem):
        c = lax.axis_index("c")
        s = lax.axis_index("s")
        pltpu.sync_copy(src_hbm.at[pl.ds(s * ES, ES)], esrc)
        pltpu.sync_copy(dst_hbm.at[pl.ds(s * ES, ES)], edst)

        for b in range(npsc):
            blk = c * npsc + b
            live = blk < nblocks
            base = jnp.minimum(blk, nblocks - 1) * rows_b

            @pl.when(live)
            def _():
                pltpu.sync_copy(g_hbm.at[pl.ds(base + s * rpw, rpw)],
                                acc.at[pl.ds(s * rpw, rpw)])
            plsc.subcore_barrier()

            def scan_body(i, cnt):
                dv = edst[pl.ds(i * 16, 16)]
                sv = esrc[pl.ds(i * 16, 16)]
                m = (dv >= base) & (dv < base + rows_b)
                plsc.store_compressed(srcl.at[pl.ds(cnt, 16)], sv, mask=m)
                plsc.store_compressed(dstl.at[pl.ds(cnt, 16)], dv - base,
                                      mask=m)
                return cnt + jnp.sum(m.astype(jnp.int32))

            cnt = jnp.where(live, lax.fori_loop(0, nscan, scan_body, 0), 0)

            nch = (cnt + k - 1) // k

            def pad_body(j, _):
                off = cnt + j * 16
                srcl[pl.ds(off, 16)] = jnp.zeros((16,), jnp.int32)
                dstl[pl.ds(off, 16)] = jnp.full((16,), rows_b, jnp.int32)
                return 0

            lax.fori_loop(0, (nch * k - cnt + 15) // 16, pad_body, 0)

            def chunk_body(ci, _):
                pltpu.async_copy(g_hbm.at[srcl.at[pl.ds(ci * k, k)]],
                                 gbuf, gsem).wait()
                for j in range(k // 16):
                    stag[pl.ds(j * 16, 16)] = dstl[pl.ds(ci * k + j * 16, 16)]
                pltpu.sync_copy(gbuf, acc.at[stag], add=True)
                return 0

            lax.fori_loop(0, nch, chunk_body, 0)
            plsc.subcore_barrier()

            @pl.when(live)
            def _():
                pltpu.sync_copy(acc.at[pl.ds(s * rpw, rpw)],
                                out_hbm.at[pl.ds(base + s * rpw, rpw)])

    return prop


_prop16 = _make_prop(16, 2000, 128)
_prop480 = _make_prop(480, 2000, 64)
_prop960 = _make_prop(960, 2000, 32)
_prop1920 = _make_prop(1920, 400, 16)


BMF = 200
BM = 400


def _feat_body(x_ref, deg_ref, wf1, bf1, wf2, bf2, wf3, bf3, g0_ref):
    xb = x_ref[...]
    f2 = jnp.maximum(jnp.dot(xb[:, :21], wf2[...],
                             preferred_element_type=F32) + bf2[...], 0.0)
    f1 = jnp.maximum(jnp.dot(xb[:, 21:6165], wf1[...],
                             preferred_element_type=F32) + bf1[...], 0.0)
    f3 = jnp.maximum(jnp.dot(xb[:, 6165:], wf3[...],
                             preferred_element_type=F32) + bf3[...], 0.0)
    dinv = lax.rsqrt(deg_ref[...])
    feat = jnp.concatenate([f2, f1, f3, jnp.zeros((BMF, 11), F32)], axis=1)
    g0_ref[...] = feat * dinv


def _feat(x, deg, wf1, bf1, wf2, bf2, wf3, bf3):
    full = lambda r, c: pl.BlockSpec((r, c), lambda i: (0, 0))
    return pl.pallas_call(
        _feat_body,
        grid=(N // BMF,),
        in_specs=[
            pl.BlockSpec((BMF, 6485), lambda i: (i, 0)),
            pl.BlockSpec((BMF, 1), lambda i: (i, 0)),
            full(6144, 128), full(1, 128),
            full(21, 21), full(1, 21),
            full(320, 320), full(1, 320),
        ],
        out_specs=pl.BlockSpec((BMF, 480), lambda i: (i, 0)),
        out_shape=jax.ShapeDtypeStruct((N, 480), F32),
    )(x, deg, wf1, bf1, wf2, bf2, wf3, bf3)


def _mid1_body(s0_ref, deg_ref, wp1, bp1, wa1, ba1, g1_ref):
    dinv = lax.rsqrt(deg_ref[...])
    pf = s0_ref[...][:, :469] * dinv
    xh = jnp.maximum(jnp.dot(pf, wp1[...], preferred_element_type=F32)
                     + bp1[...], 0.0)
    yh = jnp.maximum(jnp.dot(pf, wa1[...], preferred_element_type=F32)
                     + ba1[...], 0.0)
    g1 = jnp.concatenate([xh, yh, jnp.zeros((BM, 22), F32)], axis=1)
    g1_ref[...] = g1 * dinv


def _mid1(s0, deg, wp1, bp1, wa1, ba1):
    full = lambda r, c: pl.BlockSpec((r, c), lambda i: (0, 0))
    return pl.pallas_call(
        _mid1_body,
        grid=(N // BM,),
        in_specs=[
            pl.BlockSpec((BM, 480), lambda i: (i, 0)),
            pl.BlockSpec((BM, 1), lambda i: (i, 0)),
            full(469, 469), full(1, 469),
            full(469, 469), full(1, 469),
        ],
        out_specs=pl.BlockSpec((BM, 960), lambda i: (i, 0)),
        out_shape=jax.ShapeDtypeStruct((N, 960), F32),
    )(s0, deg, wp1, bp1, wa1, ba1)


def _mid2_body(s1_ref, deg_ref, wp2, bp2, wa2, ba2, g2_ref):
    dinv = lax.rsqrt(deg_ref[...])
    s1 = s1_ref[...]
    tx = s1[:, :469] * dinv
    ty = s1[:, 469:938] * dinv
    xh = jnp.maximum(jnp.dot(tx, wp2[...], preferred_element_type=F32)
                     + bp2[...], 0.0)
    yh = jnp.maximum(jnp.dot(ty, wa2[...], preferred_element_type=F32)
                     + ba2[...], 0.0)
    g2 = jnp.concatenate([xh, yh, jnp.zeros((BM, 44), F32)], axis=1)
    g2_ref[...] = g2 * dinv


def _mid2(s1, deg, wp2, bp2, wa2, ba2):
    full = lambda r, c: pl.BlockSpec((r, c), lambda i: (0, 0))
    return pl.pallas_call(
        _mid2_body,
        grid=(N // BM,),
        in_specs=[
            pl.BlockSpec((BM, 960), lambda i: (i, 0)),
            pl.BlockSpec((BM, 1), lambda i: (i, 0)),
            full(469, 938), full(1, 938),
            full(469, 938), full(1, 938),
        ],
        out_specs=pl.BlockSpec((BM, 1920), lambda i: (i, 0)),
        out_shape=jax.ShapeDtypeStruct((N, 1920), F32),
    )(s1, deg, wp2, bp2, wa2, ba2)


def _final_body(s2_ref, deg_ref, batch_ref, wp3, bp3, wg1, bg1, gam, bet,
                wg2, bg2, out_ref, sums, cnts):
    i = pl.program_id(0)
    nblk = pl.num_programs(0)

    @pl.when(i == 0)
    def _():
        sums[...] = jnp.zeros_like(sums)
        cnts[...] = jnp.zeros_like(cnts)

    dinv = lax.rsqrt(deg_ref[...])
    u = s2_ref[...][:, :1876] * dinv
    z = jnp.maximum(jnp.dot(u, wp3[...], preferred_element_type=F32)
                    + bp3[...], 0.0)
    seg = batch_ref[0]
    oh = (lax.broadcasted_iota(jnp.int32, (32, BM), 0) == seg).astype(F32)
    sums[...] += jnp.dot(oh, z, preferred_element_type=F32)
    cnts[...] += jnp.sum(oh, axis=1, keepdims=True)

    @pl.when(i == nblk - 1)
    def _():
        pooled = sums[...] / jnp.maximum(cnts[...], 1.0)
        h = jnp.dot(pooled, wg1[...], preferred_element_type=F32) + bg1[...]
        mu = jnp.mean(h, axis=0, keepdims=True)
        var = jnp.mean((h - mu) ** 2, axis=0, keepdims=True)
        h = (h - mu) * lax.rsqrt(var + 1e-5) * gam[...] + bet[...]
        h = jnp.maximum(h, 0.0)
        o = jnp.dot(h, wg2[...], preferred_element_type=F32) + bg2[...]
        out_ref[...] = jax.nn.sigmoid(o)


def _final(s2, deg, batch3d, wp3, bp3, wg1, bg1, gam, bet, wg2, bg2):
    full = lambda r, c: pl.BlockSpec((r, c), lambda i: (0, 0))
    return pl.pallas_call(
        _final_body,
        grid=(N // BM,),
        in_specs=[
            pl.BlockSpec((BM, 1920), lambda i: (i, 0)),
            pl.BlockSpec((BM, 1), lambda i: (i, 0)),
            pl.BlockSpec((1, 1, BM), lambda i: (i, 0, 0)),
            full(1876, 1876), full(1, 1876),
            full(1876, 1024), full(1, 1024),
            full(1, 1024), full(1, 1024),
            full(1024, 486), full(1, 486),
        ],
        out_specs=pl.BlockSpec((32, 486), lambda i: (0, 0)),
        out_shape=jax.ShapeDtypeStruct((32, 486), F32),
        scratch_shapes=[
            pltpu.VMEM((32, 1876), F32),
            pltpu.VMEM((32, 1), F32),
        ],
    )(s2, deg, batch3d, wp3, bp3, wg1, bg1, gam, bet, wg2, bg2)


def kernel(x, edge_index, batch, W_f1, b_f1, W_f2, b_f2, W_f3, b_f3,
           W_p1, b_p1, W_p2, b_p2, W_a1, b_a1, W_a2, b_a2, W_p3, b_p3,
           W_g1, b_g1, gamma, beta, W_g2, b_g2):
    src = edge_index[0]
    dst = edge_index[1]
    row = lambda v: v.reshape(1, -1)

    ones_g = jnp.ones((N, 16), F32)
    deg = _prop16(ones_g, src, dst)[:, :1]

    g0 = _feat(x, deg, W_f1, row(b_f1), W_f2, row(b_f2), W_f3, row(b_f3))
    s0 = _prop480(g0, src, dst)
    g1 = _mid1(s0, deg, W_p1, row(b_p1), W_a1, row(b_a1))
    s1 = _prop960(g1, src, dst)
    g2 = _mid2(s1, deg, W_p2, row(b_p2), W_a2, row(b_a2))
    s2 = _prop1920(g2, src, dst)
    out = _final(s2, deg, batch.reshape(N // BM, 1, BM), W_p3, row(b_p3),
                 W_g1, row(b_g1), row(gamma), row(beta), W_g2, row(b_g2))
    return out


# speedup vs baseline: 4.9256x; 4.9256x over previous
import functools

import jax
import jax.numpy as jnp
from jax import lax
from jax.experimental import pallas as pl
from jax.experimental.pallas import tpu as pltpu
from jax.experimental.pallas import tpu_sc as plsc

N = 10000
E = 160000
NS = 16
ES = E // NS
K = 80
F32 = jnp.float32


def _make_prop(Dc, two):
    assert Dc % 16 == 0 and N * Dc * 4 <= 8 * 1024 * 1024
    mesh = plsc.VectorSubcoreMesh(core_axis_name="c", subcore_axis_name="s")
    chunk_t = jax.ShapeDtypeStruct((N, Dc), F32)

    scratch = [
        pltpu.VMEM((ES,), jnp.int32),
        pltpu.VMEM((ES,), jnp.int32),
        pltpu.VMEM((K,), jnp.int32),
        pltpu.VMEM((K,), jnp.int32),
        pltpu.VMEM((K, Dc), F32),
        pltpu.VMEM_SHARED((N, Dc), F32),
        pltpu.SemaphoreType.DMA,
    ]

    def run(g_hbm, out_hbm, s, esrc, edst, ssrc, sdst, gbuf, acc, gsem):
        @pl.when(s < 15)
        def _():
            pltpu.sync_copy(g_hbm.at[pl.ds(s * 624, 624)],
                            acc.at[pl.ds(s * 624, 624)])

        @pl.when(s == 15)
        def _():
            pltpu.sync_copy(g_hbm.at[pl.ds(9360, 640)],
                            acc.at[pl.ds(9360, 640)])

        plsc.subcore_barrier()

        def group(ci, _):
            for j in range(K // 16):
                ssrc[pl.ds(j * 16, 16)] = esrc[pl.ds(ci * K + j * 16, 16)]
                sdst[pl.ds(j * 16, 16)] = edst[pl.ds(ci * K + j * 16, 16)]
            pltpu.async_copy(g_hbm.at[ssrc], gbuf, gsem).wait()
            pltpu.sync_copy(gbuf, acc.at[sdst], add=True)
            return 0

        lax.fori_loop(0, ES // K, group, 0)
        plsc.subcore_barrier()

        @pl.when(s < 15)
        def _():
            pltpu.sync_copy(acc.at[pl.ds(s * 624, 624)],
                            out_hbm.at[pl.ds(s * 624, 624)])

        @pl.when(s == 15)
        def _():
            pltpu.sync_copy(acc.at[pl.ds(9360, 640)],
                            out_hbm.at[pl.ds(9360, 640)])

    if two:
        @functools.partial(
            pl.kernel, out_type=[chunk_t, chunk_t], mesh=mesh,
            scratch_types=scratch)
        def prop(ga, gb, src_hbm, dst_hbm, outa, outb,
                 esrc, edst, ssrc, sdst, gbuf, acc, gsem):
            c = lax.axis_index("c")
            s = lax.axis_index("s")
            pltpu.sync_copy(src_hbm.at[pl.ds(s * ES, ES)], esrc)
            pltpu.sync_copy(dst_hbm.at[pl.ds(s * ES, ES)], edst)

            @pl.when(c == 0)
            def _():
                run(ga, outa, s, esrc, edst, ssrc, sdst, gbuf, acc, gsem)

            @pl.when(c == 1)
            def _():
                run(gb, outb, s, esrc, edst, ssrc, sdst, gbuf, acc, gsem)
    else:
        @functools.partial(
            pl.kernel, out_type=chunk_t, mesh=mesh, scratch_types=scratch)
        def prop(ga, src_hbm, dst_hbm, outa,
                 esrc, edst, ssrc, sdst, gbuf, acc, gsem):
            c = lax.axis_index("c")
            s = lax.axis_index("s")

            @pl.when(c == 0)
            def _():
                pltpu.sync_copy(src_hbm.at[pl.ds(s * ES, ES)], esrc)
                pltpu.sync_copy(dst_hbm.at[pl.ds(s * ES, ES)], edst)
                run(ga, outa, s, esrc, edst, ssrc, sdst, gbuf, acc, gsem)

    return prop


_prop1 = _make_prop(128, two=False)
_prop2 = _make_prop(128, two=True)


BMF = 200
BM = 400


def _feat_body(x_ref, deg_ref, wf1, bf1, wf2, bf2, wf3, bf3, g0_ref):
    xb = x_ref[...]
    f2 = jnp.maximum(jnp.dot(xb[:, :21], wf2[...],
                             preferred_element_type=F32) + bf2[...], 0.0)
    f1 = jnp.maximum(jnp.dot(xb[:, 21:6165], wf1[...],
                             preferred_element_type=F32) + bf1[...], 0.0)
    f3 = jnp.maximum(jnp.dot(xb[:, 6165:], wf3[...],
                             preferred_element_type=F32) + bf3[...], 0.0)
    dinv = lax.rsqrt(deg_ref[...])
    feat = jnp.concatenate([f2, f1, f3, jnp.zeros((BMF, 43), F32)], axis=1)
    g0_ref[...] = feat * dinv


def _feat(x, deg, wf1, bf1, wf2, bf2, wf3, bf3):
    full = lambda r, c: pl.BlockSpec((r, c), lambda i: (0, 0))
    return pl.pallas_call(
        _feat_body,
        grid=(N // BMF,),
        in_specs=[
            pl.BlockSpec((BMF, 6485), lambda i: (i, 0)),
            pl.BlockSpec((BMF, 1), lambda i: (i, 0)),
            full(6144, 128), full(1, 128),
            full(21, 21), full(1, 21),
            full(320, 320), full(1, 320),
        ],
        out_specs=pl.BlockSpec((BMF, 512), lambda i: (i, 0)),
        out_shape=jax.ShapeDtypeStruct((N, 512), F32),
    )(x, deg, wf1, bf1, wf2, bf2, wf3, bf3)


def _mid1_body(s0_ref, deg_ref, wp1, bp1, wa1, ba1, g1_ref):
    dinv = lax.rsqrt(deg_ref[...])
    pf = s0_ref[...][:, :469] * dinv
    xh = jnp.maximum(jnp.dot(pf, wp1[...], preferred_element_type=F32)
                     + bp1[...], 0.0)
    yh = jnp.maximum(jnp.dot(pf, wa1[...], preferred_element_type=F32)
                     + ba1[...], 0.0)
    g1 = jnp.concatenate([xh, yh, jnp.zeros((BM, 86), F32)], axis=1)
    g1_ref[...] = g1 * dinv


def _mid1(s0, deg, wp1, bp1, wa1, ba1):
    full = lambda r, c: pl.BlockSpec((r, c), lambda i: (0, 0))
    return pl.pallas_call(
        _mid1_body,
        grid=(N // BM,),
        in_specs=[
            pl.BlockSpec((BM, 512), lambda i: (i, 0)),
            pl.BlockSpec((BM, 1), lambda i: (i, 0)),
            full(469, 469), full(1, 469),
            full(469, 469), full(1, 469),
        ],
        out_specs=pl.BlockSpec((BM, 1024), lambda i: (i, 0)),
        out_shape=jax.ShapeDtypeStruct((N, 1024), F32),
    )(s0, deg, wp1, bp1, wa1, ba1)


def _mid2_body(s1_ref, deg_ref, wp2, bp2, wa2, ba2, g2_ref):
    dinv = lax.rsqrt(deg_ref[...])
    s1 = s1_ref[...]
    tx = s1[:, :469] * dinv
    ty = s1[:, 469:938] * dinv
    xh = jnp.maximum(jnp.dot(tx, wp2[...], preferred_element_type=F32)
                     + bp2[...], 0.0)
    yh = jnp.maximum(jnp.dot(ty, wa2[...], preferred_element_type=F32)
                     + ba2[...], 0.0)
    g2 = jnp.concatenate([xh, yh, jnp.zeros((BM, 172), F32)], axis=1)
    g2_ref[...] = g2 * dinv


def _mid2(s1, deg, wp2, bp2, wa2, ba2):
    full = lambda r, c: pl.BlockSpec((r, c), lambda i: (0, 0))
    return pl.pallas_call(
        _mid2_body,
        grid=(N // BM,),
        in_specs=[
            pl.BlockSpec((BM, 1024), lambda i: (i, 0)),
            pl.BlockSpec((BM, 1), lambda i: (i, 0)),
            full(469, 938), full(1, 938),
            full(469, 938), full(1, 938),
        ],
        out_specs=pl.BlockSpec((BM, 2048), lambda i: (i, 0)),
        out_shape=jax.ShapeDtypeStruct((N, 2048), F32),
    )(s1, deg, wp2, bp2, wa2, ba2)


def _final_body(s2_ref, deg_ref, batch_ref, wp3, bp3, wg1, bg1, gam, bet,
                wg2, bg2, out_ref, sums, cnts):
    i = pl.program_id(0)
    nblk = pl.num_programs(0)

    @pl.when(i == 0)
    def _():
        sums[...] = jnp.zeros_like(sums)
        cnts[...] = jnp.zeros_like(cnts)

    dinv = lax.rsqrt(deg_ref[...])
    u = s2_ref[...][:, :1876] * dinv
    z = jnp.maximum(jnp.dot(u, wp3[...], preferred_element_type=F32)
                    + bp3[...], 0.0)
    seg = batch_ref[0]
    oh = (lax.broadcasted_iota(jnp.int32, (32, BM), 0) == seg).astype(F32)
    sums[...] += jnp.dot(oh, z, preferred_element_type=F32)
    cnts[...] += jnp.sum(oh, axis=1, keepdims=True)

    @pl.when(i == nblk - 1)
    def _():
        pooled = sums[...] / jnp.maximum(cnts[...], 1.0)
        h = jnp.dot(pooled, wg1[...], preferred_element_type=F32) + bg1[...]
        mu = jnp.mean(h, axis=0, keepdims=True)
        var = jnp.mean((h - mu) ** 2, axis=0, keepdims=True)
        h = (h - mu) * lax.rsqrt(var + 1e-5) * gam[...] + bet[...]
        h = jnp.maximum(h, 0.0)
        o = jnp.dot(h, wg2[...], preferred_element_type=F32) + bg2[...]
        out_ref[...] = jax.nn.sigmoid(o)


def _final(s2, deg, batch3d, wp3, bp3, wg1, bg1, gam, bet, wg2, bg2):
    full = lambda r, c: pl.BlockSpec((r, c), lambda i: (0, 0))
    return pl.pallas_call(
        _final_body,
        grid=(N // BM,),
        in_specs=[
            pl.BlockSpec((BM, 2048), lambda i: (i, 0)),
            pl.BlockSpec((BM, 1), lambda i: (i, 0)),
            pl.BlockSpec((1, 1, BM), lambda i: (i, 0, 0)),
            full(1876, 1876), full(1, 1876),
            full(1876, 1024), full(1, 1024),
            full(1, 1024), full(1, 1024),
            full(1024, 486), full(1, 486),
        ],
        out_specs=pl.BlockSpec((32, 486), lambda i: (0, 0)),
        out_shape=jax.ShapeDtypeStruct((32, 486), F32),
        scratch_shapes=[
            pltpu.VMEM((32, 1876), F32),
            pltpu.VMEM((32, 1), F32),
        ],
    )(s2, deg, batch3d, wp3, bp3, wg1, bg1, gam, bet, wg2, bg2)


def kernel(x, edge_index, batch, W_f1, b_f1, W_f2, b_f2, W_f3, b_f3,
           W_p1, b_p1, W_p2, b_p2, W_a1, b_a1, W_a2, b_a2, W_p3, b_p3,
           W_g1, b_g1, gamma, beta, W_g2, b_g2):
    src = edge_index[0]
    dst = edge_index[1]
    row = lambda v: v.reshape(1, -1)

    deg = _prop1(jnp.ones((N, 128), F32), src, dst)[:, :1]

    def prop(g, width):
        half = width // 2
        nc = half // 128
        parts = [_prop2(g[:, i * 128:(i + 1) * 128],
                        g[:, half + i * 128:half + (i + 1) * 128], src, dst)
                 for i in range(nc)]
        return jnp.concatenate([ab[0] for ab in parts]
                               + [ab[1] for ab in parts], axis=1)

    g0 = _feat(x, deg, W_f1, row(b_f1), W_f2, row(b_f2), W_f3, row(b_f3))
    s0 = prop(g0, 512)
    g1 = _mid1(s0, deg, W_p1, row(b_p1), W_a1, row(b_a1))
    s1 = prop(g1, 1024)
    g2 = _mid2(s1, deg, W_p2, row(b_p2), W_a2, row(b_a2))
    s2 = prop(g2, 2048)

    out = _final(s2, deg, batch.reshape(N // BM, 1, BM), W_p3, row(b_p3),
                 W_g1, row(b_g1), row(gamma), row(beta), W_g2, row(b_g2))
    return out


# speedup vs baseline: 4.9259x; 1.0001x over previous
import functools

import jax
import jax.numpy as jnp
from jax import lax
from jax.experimental import pallas as pl
from jax.experimental.pallas import tpu as pltpu
from jax.experimental.pallas import tpu_sc as plsc

N = 10000
E = 160000
NS = 16
ES = E // NS
K = 80
F32 = jnp.float32


def _make_prop(Dc, two):
    assert Dc % 128 == 0 and N * Dc * 4 <= 8 * 1024 * 1024
    mesh = plsc.VectorSubcoreMesh(core_axis_name="c", subcore_axis_name="s")
    chunk_t = jax.ShapeDtypeStruct((N, Dc), F32)

    scratch = [
        pltpu.VMEM((ES,), jnp.int32),
        pltpu.VMEM((ES,), jnp.int32),
        pltpu.VMEM((K,), jnp.int32),
        pltpu.VMEM((K,), jnp.int32),
        pltpu.VMEM((K, Dc), F32),
        pltpu.VMEM_SHARED((N, Dc), F32),
        pltpu.SemaphoreType.DMA,
    ]

    def run(g_hbm, out_hbm, s, esrc, edst, ssrc, sdst, gbuf, acc, gsem):
        @pl.when(s < 15)
        def _():
            pltpu.sync_copy(g_hbm.at[pl.ds(s * 624, 624)],
                            acc.at[pl.ds(s * 624, 624)])

        @pl.when(s == 15)
        def _():
            pltpu.sync_copy(g_hbm.at[pl.ds(9360, 640)],
                            acc.at[pl.ds(9360, 640)])

        plsc.subcore_barrier()

        def group(ci, _):
            for j in range(K // 16):
                ssrc[pl.ds(j * 16, 16)] = esrc[pl.ds(ci * K + j * 16, 16)]
                sdst[pl.ds(j * 16, 16)] = edst[pl.ds(ci * K + j * 16, 16)]
            pltpu.async_copy(g_hbm.at[ssrc], gbuf, gsem).wait()
            pltpu.sync_copy(gbuf, acc.at[sdst], add=True)
            return 0

        lax.fori_loop(0, ES // K, group, 0)
        plsc.subcore_barrier()

        @pl.when(s < 15)
        def _():
            pltpu.sync_copy(acc.at[pl.ds(s * 624, 624)],
                            out_hbm.at[pl.ds(s * 624, 624)])

        @pl.when(s == 15)
        def _():
            pltpu.sync_copy(acc.at[pl.ds(9360, 640)],
                            out_hbm.at[pl.ds(9360, 640)])

    if two:
        @functools.partial(
            pl.kernel, out_type=[chunk_t, chunk_t], mesh=mesh,
            scratch_types=scratch)
        def prop(ga, gb, src_hbm, dst_hbm, outa, outb,
                 esrc, edst, ssrc, sdst, gbuf, acc, gsem):
            c = lax.axis_index("c")
            s = lax.axis_index("s")
            pltpu.sync_copy(src_hbm.at[pl.ds(s * ES, ES)], esrc)
            pltpu.sync_copy(dst_hbm.at[pl.ds(s * ES, ES)], edst)

            @pl.when(c == 0)
            def _():
                run(ga, outa, s, esrc, edst, ssrc, sdst, gbuf, acc, gsem)

            @pl.when(c == 1)
            def _():
                run(gb, outb, s, esrc, edst, ssrc, sdst, gbuf, acc, gsem)
    else:
        @functools.partial(
            pl.kernel, out_type=chunk_t, mesh=mesh, scratch_types=scratch)
        def prop(ga, src_hbm, dst_hbm, outa,
                 esrc, edst, ssrc, sdst, gbuf, acc, gsem):
            c = lax.axis_index("c")
            s = lax.axis_index("s")

            @pl.when(c == 0)
            def _():
                pltpu.sync_copy(src_hbm.at[pl.ds(s * ES, ES)], esrc)
                pltpu.sync_copy(dst_hbm.at[pl.ds(s * ES, ES)], edst)
                run(ga, outa, s, esrc, edst, ssrc, sdst, gbuf, acc, gsem)

    return prop


_prop1 = _make_prop(128, two=False)
_prop2 = _make_prop(128, two=True)


BMF = 200
BM = 400


def _feat_body(x_ref, deg_ref, wf1, bf1, wf2, bf2, wf3, bf3, g0_ref):
    xb = x_ref[...]
    f2 = jnp.maximum(jnp.dot(xb[:, :21], wf2[...],
                             preferred_element_type=F32) + bf2[...], 0.0)
    f1 = jnp.maximum(jnp.dot(xb[:, 21:6165], wf1[...],
                             preferred_element_type=F32) + bf1[...], 0.0)
    f3 = jnp.maximum(jnp.dot(xb[:, 6165:], wf3[...],
                             preferred_element_type=F32) + bf3[...], 0.0)
    dinv = lax.rsqrt(deg_ref[...])
    feat = jnp.concatenate([f2, f1, f3, jnp.zeros((BMF, 43), F32)], axis=1)
    g0_ref[...] = feat * dinv


def _feat(x, deg, wf1, bf1, wf2, bf2, wf3, bf3):
    full = lambda r, c: pl.BlockSpec((r, c), lambda i: (0, 0))
    return pl.pallas_call(
        _feat_body,
        grid=(N // BMF,),
        in_specs=[
            pl.BlockSpec((BMF, 6485), lambda i: (i, 0)),
            pl.BlockSpec((BMF, 1), lambda i: (i, 0)),
            full(6144, 128), full(1, 128),
            full(21, 21), full(1, 21),
            full(320, 320), full(1, 320),
        ],
        out_specs=pl.BlockSpec((BMF, 512), lambda i: (i, 0)),
        out_shape=jax.ShapeDtypeStruct((N, 512), F32),
    )(x, deg, wf1, bf1, wf2, bf2, wf3, bf3)


def _mid1_body(s0_ref, deg_ref, wp1, bp1, wa1, ba1, g1_ref):
    dinv = lax.rsqrt(deg_ref[...])
    pf = s0_ref[...][:, :469] * dinv
    xh = jnp.maximum(jnp.dot(pf, wp1[...], preferred_element_type=F32)
                     + bp1[...], 0.0)
    yh = jnp.maximum(jnp.dot(pf, wa1[...], preferred_element_type=F32)
                     + ba1[...], 0.0)
    g1 = jnp.concatenate([xh, yh, jnp.zeros((BM, 86), F32)], axis=1)
    g1_ref[...] = g1 * dinv


def _mid1(s0, deg, wp1, bp1, wa1, ba1):
    full = lambda r, c: pl.BlockSpec((r, c), lambda i: (0, 0))
    return pl.pallas_call(
        _mid1_body,
        grid=(N // BM,),
        in_specs=[
            pl.BlockSpec((BM, 512), lambda i: (i, 0)),
            pl.BlockSpec((BM, 1), lambda i: (i, 0)),
            full(469, 469), full(1, 469),
            full(469, 469), full(1, 469),
        ],
        out_specs=pl.BlockSpec((BM, 1024), lambda i: (i, 0)),
        out_shape=jax.ShapeDtypeStruct((N, 1024), F32),
    )(s0, deg, wp1, bp1, wa1, ba1)


def _mid2_body(s1_ref, deg_ref, wp2, bp2, wa2, ba2, g2_ref):
    dinv = lax.rsqrt(deg_ref[...])
    s1 = s1_ref[...]
    tx = s1[:, :469] * dinv
    ty = s1[:, 469:938] * dinv
    xh = jnp.maximum(jnp.dot(tx, wp2[...], preferred_element_type=F32)
                     + bp2[...], 0.0)
    yh = jnp.maximum(jnp.dot(ty, wa2[...], preferred_element_type=F32)
                     + ba2[...], 0.0)
    g2 = jnp.concatenate([xh, yh, jnp.zeros((BM, 172), F32)], axis=1)
    g2_ref[...] = g2 * dinv


def _mid2(s1, deg, wp2, bp2, wa2, ba2):
    full = lambda r, c: pl.BlockSpec((r, c), lambda i: (0, 0))
    return pl.pallas_call(
        _mid2_body,
        grid=(N // BM,),
        in_specs=[
            pl.BlockSpec((BM, 1024), lambda i: (i, 0)),
            pl.BlockSpec((BM, 1), lambda i: (i, 0)),
            full(469, 938), full(1, 938),
            full(469, 938), full(1, 938),
        ],
        out_specs=pl.BlockSpec((BM, 2048), lambda i: (i, 0)),
        out_shape=jax.ShapeDtypeStruct((N, 2048), F32),
    )(s1, deg, wp2, bp2, wa2, ba2)


def _final_body(s2_ref, deg_ref, batch_ref, wp3, bp3, wg1, bg1, gam, bet,
                wg2, bg2, out_ref, sums, cnts):
    i = pl.program_id(0)
    nblk = pl.num_programs(0)

    @pl.when(i == 0)
    def _():
        sums[...] = jnp.zeros_like(sums)
        cnts[...] = jnp.zeros_like(cnts)

    dinv = lax.rsqrt(deg_ref[...])
    u = s2_ref[...][:, :1876] * dinv
    z = jnp.maximum(jnp.dot(u, wp3[...], preferred_element_type=F32)
                    + bp3[...], 0.0)
    seg = batch_ref[0]
    oh = (lax.broadcasted_iota(jnp.int32, (32, BM), 0) == seg).astype(F32)
    sums[...] += jnp.dot(oh, z, preferred_element_type=F32)
    cnts[...] += jnp.sum(oh, axis=1, keepdims=True)

    @pl.when(i == nblk - 1)
    def _():
        pooled = sums[...] / jnp.maximum(cnts[...], 1.0)
        h = jnp.dot(pooled, wg1[...], preferred_element_type=F32) + bg1[...]
        mu = jnp.mean(h, axis=0, keepdims=True)
        var = jnp.mean((h - mu) ** 2, axis=0, keepdims=True)
        h = (h - mu) * lax.rsqrt(var + 1e-5) * gam[...] + bet[...]
        h = jnp.maximum(h, 0.0)
        o = jnp.dot(h, wg2[...], preferred_element_type=F32) + bg2[...]
        out_ref[...] = jax.nn.sigmoid(o)


def _final(s2, deg, batch3d, wp3, bp3, wg1, bg1, gam, bet, wg2, bg2):
    full = lambda r, c: pl.BlockSpec((r, c), lambda i: (0, 0))
    return pl.pallas_call(
        _final_body,
        grid=(N // BM,),
        in_specs=[
            pl.BlockSpec((BM, 2048), lambda i: (i, 0)),
            pl.BlockSpec((BM, 1), lambda i: (i, 0)),
            pl.BlockSpec((1, 1, BM), lambda i: (i, 0, 0)),
            full(1876, 1876), full(1, 1876),
            full(1876, 1024), full(1, 1024),
            full(1, 1024), full(1, 1024),
            full(1024, 486), full(1, 486),
        ],
        out_specs=pl.BlockSpec((32, 486), lambda i: (0, 0)),
        out_shape=jax.ShapeDtypeStruct((32, 486), F32),
        scratch_shapes=[
            pltpu.VMEM((32, 1876), F32),
            pltpu.VMEM((32, 1), F32),
        ],
    )(s2, deg, batch3d, wp3, bp3, wg1, bg1, gam, bet, wg2, bg2)


def kernel(x, edge_index, batch, W_f1, b_f1, W_f2, b_f2, W_f3, b_f3,
           W_p1, b_p1, W_p2, b_p2, W_a1, b_a1, W_a2, b_a2, W_p3, b_p3,
           W_g1, b_g1, gamma, beta, W_g2, b_g2):
    src = edge_index[0]
    dst = edge_index[1]
    row = lambda v: v.reshape(1, -1)

    deg = _prop1(jnp.ones((N, 128), F32), src, dst)[:, :1]

    def prop(g, width):
        half = width // 2
        nc = half // 128
        parts = [_prop2(g[:, i * 128:(i + 1) * 128],
                        g[:, half + i * 128:half + (i + 1) * 128], src, dst)
                 for i in range(nc)]
        return jnp.concatenate([ab[0] for ab in parts]
                               + [ab[1] for ab in parts], axis=1)

    g0 = _feat(x, deg, W_f1, row(b_f1), W_f2, row(b_f2), W_f3, row(b_f3))
    s0 = prop(g0, 512)
    g1 = _mid1(s0, deg, W_p1, row(b_p1), W_a1, row(b_a1))
    s1 = prop(g1, 1024)
    g2 = _mid2(s1, deg, W_p2, row(b_p2), W_a2, row(b_a2))
    s2 = prop(g2, 2048)

    out = _final(s2, deg, batch.reshape(N // BM, 1, BM), W_p3, row(b_p3),
                 W_g1, row(b_g1), row(gamma), row(beta), W_g2, row(b_g2))
    return out


# speedup vs baseline: 6.9175x; 1.4043x over previous
import functools

import jax
import jax.numpy as jnp
from jax import lax
from jax.experimental import pallas as pl
from jax.experimental.pallas import tpu as pltpu
from jax.experimental.pallas import tpu_sc as plsc

N = 10000
E = 160000
NS = 16
ES = E // NS
K = 80
G = ES // K
ESD = E // (2 * NS)
KD = 40
GD = ESD // KD
F32 = jnp.float32


def _make_prop(Dc, two):
    assert Dc % 128 == 0 and N * Dc * 4 <= 8 * 1024 * 1024
    mesh = plsc.VectorSubcoreMesh(core_axis_name="c", subcore_axis_name="s")
    chunk_t = jax.ShapeDtypeStruct((N, Dc), F32)

    scratch = [
        pltpu.VMEM((2, K), jnp.int32),
        pltpu.VMEM((2, K), jnp.int32),
        pltpu.VMEM((K, Dc), F32),
        pltpu.VMEM((K, Dc), F32),
        pltpu.VMEM_SHARED((N, Dc), F32),
        pltpu.SemaphoreType.DMA,
        pltpu.SemaphoreType.DMA,
    ]

    def run(g_hbm, eidx_hbm, out_hbm, s, sidx0, sidx1, gbuf0, gbuf1, acc,
            gsem0, gsem1):
        @pl.when(s < 15)
        def _():
            pltpu.sync_copy(g_hbm.at[pl.ds(s * 624, 624)],
                            acc.at[pl.ds(s * 624, 624)])

        @pl.when(s == 15)
        def _():
            pltpu.sync_copy(g_hbm.at[pl.ds(9360, 640)],
                            acc.at[pl.ds(9360, 640)])

        plsc.subcore_barrier()

        pltpu.sync_copy(eidx_hbm.at[s].at[0], sidx0)
        pltpu.async_copy(g_hbm.at[sidx0.at[0]], gbuf0, gsem0)

        def group(i, _):
            nxt = i + 1

            @pl.when(jnp.logical_and(nxt < G, nxt % 2 == 1))
            def _():
                pltpu.sync_copy(eidx_hbm.at[s].at[nxt], sidx1)
                pltpu.async_copy(g_hbm.at[sidx1.at[0]], gbuf1, gsem1)

            @pl.when(jnp.logical_and(nxt < G, nxt % 2 == 0))
            def _():
                pltpu.sync_copy(eidx_hbm.at[s].at[nxt], sidx0)
                pltpu.async_copy(g_hbm.at[sidx0.at[0]], gbuf0, gsem0)

            @pl.when(i % 2 == 0)
            def _():
                pltpu.make_async_copy(g_hbm.at[sidx0.at[0]], gbuf0,
                                      gsem0).wait()
                pltpu.sync_copy(gbuf0, acc.at[sidx0.at[1]], add=True)

            @pl.when(i % 2 == 1)
            def _():
                pltpu.make_async_copy(g_hbm.at[sidx1.at[0]], gbuf1,
                                      gsem1).wait()
                pltpu.sync_copy(gbuf1, acc.at[sidx1.at[1]], add=True)

            return 0

        lax.fori_loop(0, G, group, 0)
        plsc.subcore_barrier()

        @pl.when(s < 15)
        def _():
            pltpu.sync_copy(acc.at[pl.ds(s * 624, 624)],
                            out_hbm.at[pl.ds(s * 624, 624)])

        @pl.when(s == 15)
        def _():
            pltpu.sync_copy(acc.at[pl.ds(9360, 640)],
                            out_hbm.at[pl.ds(9360, 640)])

    if two:
        @functools.partial(
            pl.kernel, out_type=[chunk_t, chunk_t], mesh=mesh,
            scratch_types=scratch)
        def prop(ga, gb, eidx_hbm, outa, outb,
                 sidx0, sidx1, gbuf0, gbuf1, acc, gsem0, gsem1):
            c = lax.axis_index("c")
            s = lax.axis_index("s")

            @pl.when(c == 0)
            def _():
                run(ga, eidx_hbm, outa, s, sidx0, sidx1, gbuf0, gbuf1, acc,
                    gsem0, gsem1)

            @pl.when(c == 1)
            def _():
                run(gb, eidx_hbm, outb, s, sidx0, sidx1, gbuf0, gbuf1, acc,
                    gsem0, gsem1)
    else:
        @functools.partial(
            pl.kernel, out_type=chunk_t, mesh=mesh, scratch_types=scratch)
        def prop(ga, eidx_hbm, outa,
                 sidx0, sidx1, gbuf0, gbuf1, acc, gsem0, gsem1):
            c = lax.axis_index("c")
            s = lax.axis_index("s")

            @pl.when(c == 0)
            def _():
                run(ga, eidx_hbm, outa, s, sidx0, sidx1, gbuf0, gbuf1, acc,
                    gsem0, gsem1)

    return prop


_prop2 = _make_prop(128, two=True)


def _make_deg():
    mesh = plsc.VectorSubcoreMesh(core_axis_name="c", subcore_axis_name="s")
    out_t = jax.ShapeDtypeStruct((N, 16), F32)
    scratch = [
        pltpu.VMEM((GD, KD), jnp.int32),
        pltpu.VMEM((KD, 16), F32),
        pltpu.VMEM_SHARED((N, 16), F32),
        pltpu.SemaphoreType.DMA,
    ]

    @functools.partial(pl.kernel, out_type=[out_t, out_t], mesh=mesh,
                       scratch_types=scratch)
    def deg(ones_hbm, dst_hbm, out0, out1, edst, ones, acc, ssem):
        c = lax.axis_index("c")
        s = lax.axis_index("s")
        pltpu.sync_copy(dst_hbm.at[c].at[s], edst)
        pltpu.sync_copy(ones_hbm.at[pl.ds(0, KD)], ones)

        @pl.when(s < 15)
        def _():
            pltpu.sync_copy(ones_hbm.at[pl.ds(s * 624, 624)],
                            acc.at[pl.ds(s * 624, 624)])

        @pl.when(s == 15)
        def _():
            pltpu.sync_copy(ones_hbm.at[pl.ds(9360, 640)],
                            acc.at[pl.ds(9360, 640)])

        plsc.subcore_barrier()

        def chunk(b, _):
            for j in range(5):
                pltpu.async_copy(ones, acc.at[edst.at[b * 5 + j]], ssem,
                                 add=True)
            for j in range(5):
                pltpu.make_async_copy(ones, acc.at[edst.at[b * 5 + j]],
                                      ssem).wait()
            return 0

        lax.fori_loop(0, GD // 5, chunk, 0)
        plsc.subcore_barrier()

        @pl.when(jnp.logical_and(c == 0, s < 15))
        def _():
            pltpu.sync_copy(acc.at[pl.ds(s * 624, 624)],
                            out0.at[pl.ds(s * 624, 624)])

        @pl.when(jnp.logical_and(c == 0, s == 15))
        def _():
            pltpu.sync_copy(acc.at[pl.ds(9360, 640)],
                            out0.at[pl.ds(9360, 640)])

        @pl.when(jnp.logical_and(c == 1, s < 15))
        def _():
            pltpu.sync_copy(acc.at[pl.ds(s * 624, 624)],
                            out1.at[pl.ds(s * 624, 624)])

        @pl.when(jnp.logical_and(c == 1, s == 15))
        def _():
            pltpu.sync_copy(acc.at[pl.ds(9360, 640)],
                            out1.at[pl.ds(9360, 640)])

    return deg


_deg = _make_deg()


BMF = 200
BM = 400


def _dinv(d0_ref, d1_ref):
    return lax.rsqrt(d0_ref[...][:, :1] + d1_ref[...][:, :1] - 1.0)


def _feat_body(x_ref, d0_ref, d1_ref, wf1, bf1, wf2, bf2, wf3, bf3, g0_ref):
    xb = x_ref[...]
    f2 = jnp.maximum(jnp.dot(xb[:, :21], wf2[...],
                             preferred_element_type=F32) + bf2[...], 0.0)
    f1 = jnp.maximum(jnp.dot(xb[:, 21:6165], wf1[...],
                             preferred_element_type=F32) + bf1[...], 0.0)
    f3 = jnp.maximum(jnp.dot(xb[:, 6165:], wf3[...],
                             preferred_element_type=F32) + bf3[...], 0.0)
    feat = jnp.concatenate([f2, f1, f3, jnp.zeros((BMF, 43), F32)], axis=1)
    g0_ref[...] = feat * _dinv(d0_ref, d1_ref)


def _feat(x, d0, d1, wf1, bf1, wf2, bf2, wf3, bf3):
    full = lambda r, c: pl.BlockSpec((r, c), lambda i: (0, 0))
    return pl.pallas_call(
        _feat_body,
        grid=(N // BMF,),
        in_specs=[
            pl.BlockSpec((BMF, 6485), lambda i: (i, 0)),
            pl.BlockSpec((BMF, 16), lambda i: (i, 0)),
            pl.BlockSpec((BMF, 16), lambda i: (i, 0)),
            full(6144, 128), full(1, 128),
            full(21, 21), full(1, 21),
            full(320, 320), full(1, 320),
        ],
        out_specs=pl.BlockSpec((BMF, 512), lambda i: (i, 0)),
        out_shape=jax.ShapeDtypeStruct((N, 512), F32),
    )(x, d0, d1, wf1, bf1, wf2, bf2, wf3, bf3)


def _mid1_body(s0_ref, d0_ref, d1_ref, wp1, bp1, wa1, ba1, g1_ref):
    dinv = _dinv(d0_ref, d1_ref)
    pf = s0_ref[...][:, :469] * dinv
    xh = jnp.maximum(jnp.dot(pf, wp1[...], preferred_element_type=F32)
                     + bp1[...], 0.0)
    yh = jnp.maximum(jnp.dot(pf, wa1[...], preferred_element_type=F32)
                     + ba1[...], 0.0)
    g1 = jnp.concatenate([xh, yh, jnp.zeros((BM, 86), F32)], axis=1)
    g1_ref[...] = g1 * dinv


def _mid1(s0, d0, d1, wp1, bp1, wa1, ba1):
    full = lambda r, c: pl.BlockSpec((r, c), lambda i: (0, 0))
    return pl.pallas_call(
        _mid1_body,
        grid=(N // BM,),
        in_specs=[
            pl.BlockSpec((BM, 512), lambda i: (i, 0)),
            pl.BlockSpec((BM, 16), lambda i: (i, 0)),
            pl.BlockSpec((BM, 16), lambda i: (i, 0)),
            full(469, 469), full(1, 469),
            full(469, 469), full(1, 469),
        ],
        out_specs=pl.BlockSpec((BM, 1024), lambda i: (i, 0)),
        out_shape=jax.ShapeDtypeStruct((N, 1024), F32),
    )(s0, d0, d1, wp1, bp1, wa1, ba1)


def _mid2_body(s1_ref, d0_ref, d1_ref, wp2, bp2, wa2, ba2, g2_ref):
    dinv = _dinv(d0_ref, d1_ref)
    s1 = s1_ref[...]
    tx = s1[:, :469] * dinv
    ty = s1[:, 469:938] * dinv
    xh = jnp.maximum(jnp.dot(tx, wp2[...], preferred_element_type=F32)
                     + bp2[...], 0.0)
    yh = jnp.maximum(jnp.dot(ty, wa2[...], preferred_element_type=F32)
                     + ba2[...], 0.0)
    g2 = jnp.concatenate([xh, yh, jnp.zeros((BM, 172), F32)], axis=1)
    g2_ref[...] = g2 * dinv


def _mid2(s1, d0, d1, wp2, bp2, wa2, ba2):
    full = lambda r, c: pl.BlockSpec((r, c), lambda i: (0, 0))
    return pl.pallas_call(
        _mid2_body,
        grid=(N // BM,),
        in_specs=[
            pl.BlockSpec((BM, 1024), lambda i: (i, 0)),
            pl.BlockSpec((BM, 16), lambda i: (i, 0)),
            pl.BlockSpec((BM, 16), lambda i: (i, 0)),
            full(469, 938), full(1, 938),
            full(469, 938), full(1, 938),
        ],
        out_specs=pl.BlockSpec((BM, 2048), lambda i: (i, 0)),
        out_shape=jax.ShapeDtypeStruct((N, 2048), F32),
    )(s1, d0, d1, wp2, bp2, wa2, ba2)


def _final_body(s2_ref, d0_ref, d1_ref, batch_ref, wp3, bp3, wg1, bg1, gam,
                bet, wg2, bg2, out_ref, sums, cnts):
    i = pl.program_id(0)
    nblk = pl.num_programs(0)

    @pl.when(i == 0)
    def _():
        sums[...] = jnp.zeros_like(sums)
        cnts[...] = jnp.zeros_like(cnts)

    u = s2_ref[...][:, :1876] * _dinv(d0_ref, d1_ref)
    z = jnp.maximum(jnp.dot(u, wp3[...], preferred_element_type=F32)
                    + bp3[...], 0.0)
    seg = batch_ref[0]
    oh = (lax.broadcasted_iota(jnp.int32, (32, BM), 0) == seg).astype(F32)
    sums[...] += jnp.dot(oh, z, preferred_element_type=F32)
    cnts[...] += jnp.sum(oh, axis=1, keepdims=True)

    @pl.when(i == nblk - 1)
    def _():
        pooled = sums[...] / jnp.maximum(cnts[...], 1.0)
        h = jnp.dot(pooled, wg1[...], preferred_element_type=F32) + bg1[...]
        mu = jnp.mean(h, axis=0, keepdims=True)
        var = jnp.mean((h - mu) ** 2, axis=0, keepdims=True)
        h = (h - mu) * lax.rsqrt(var + 1e-5) * gam[...] + bet[...]
        h = jnp.maximum(h, 0.0)
        o = jnp.dot(h, wg2[...], preferred_element_type=F32) + bg2[...]
        out_ref[...] = jax.nn.sigmoid(o)


def _final(s2, d0, d1, batch3d, wp3, bp3, wg1, bg1, gam, bet, wg2, bg2):
    full = lambda r, c: pl.BlockSpec((r, c), lambda i: (0, 0))
    return pl.pallas_call(
        _final_body,
        grid=(N // BM,),
        in_specs=[
            pl.BlockSpec((BM, 2048), lambda i: (i, 0)),
            pl.BlockSpec((BM, 16), lambda i: (i, 0)),
            pl.BlockSpec((BM, 16), lambda i: (i, 0)),
            pl.BlockSpec((1, 1, BM), lambda i: (i, 0, 0)),
            full(1876, 1876), full(1, 1876),
            full(1876, 1024), full(1, 1024),
            full(1, 1024), full(1, 1024),
            full(1024, 486), full(1, 486),
        ],
        out_specs=pl.BlockSpec((32, 486), lambda i: (0, 0)),
        out_shape=jax.ShapeDtypeStruct((32, 486), F32),
        scratch_shapes=[
            pltpu.VMEM((32, 1876), F32),
            pltpu.VMEM((32, 1), F32),
        ],
    )(s2, d0, d1, batch3d, wp3, bp3, wg1, bg1, gam, bet, wg2, bg2)


def kernel(x, edge_index, batch, W_f1, b_f1, W_f2, b_f2, W_f3, b_f3,
           W_p1, b_p1, W_p2, b_p2, W_a1, b_a1, W_a2, b_a2, W_p3, b_p3,
           W_g1, b_g1, gamma, beta, W_g2, b_g2):
    eidx = jnp.stack([edge_index[0].reshape(NS, G, K),
                      edge_index[1].reshape(NS, G, K)], axis=2)
    dst4 = edge_index[1].reshape(2, NS, GD, KD)
    ones = jnp.ones((N, 16), F32)
    row = lambda v: v.reshape(1, -1)

    d0, d1 = _deg(ones, dst4)

    def prop(g, width):
        half = width // 2
        nc = half // 128
        parts = [_prop2(g[:, i * 128:(i + 1) * 128],
                        g[:, half + i * 128:half + (i + 1) * 128],
                        eidx)
                 for i in range(nc)]
        return jnp.concatenate([ab[0] for ab in parts]
                               + [ab[1] for ab in parts], axis=1)

    g0 = _feat(x, d0, d1, W_f1, row(b_f1), W_f2, row(b_f2), W_f3, row(b_f3))
    s0 = prop(g0, 512)
    g1 = _mid1(s0, d0, d1, W_p1, row(b_p1), W_a1, row(b_a1))
    s1 = prop(g1, 1024)
    g2 = _mid2(s1, d0, d1, W_p2, row(b_p2), W_a2, row(b_a2))
    s2 = prop(g2, 2048)

    out = _final(s2, d0, d1, batch.reshape(N // BM, 1, BM), W_p3, row(b_p3),
                 W_g1, row(b_g1), row(gamma), row(beta), W_g2, row(b_g2))
    return out


# speedup vs baseline: 6.9238x; 1.0009x over previous
import functools

import jax
import jax.numpy as jnp
from jax import lax
from jax.experimental import pallas as pl
from jax.experimental.pallas import tpu as pltpu
from jax.experimental.pallas import tpu_sc as plsc

N = 10000
E = 160000
NS = 16
ES = E // NS
K = 80
G = ES // K
ESD = E // (2 * NS)
KD = 40
GD = ESD // KD
F32 = jnp.float32


def _make_prop(Dc, two):
    assert Dc % 128 == 0 and N * Dc * 4 <= 8 * 1024 * 1024
    mesh = plsc.VectorSubcoreMesh(core_axis_name="c", subcore_axis_name="s")
    chunk_t = jax.ShapeDtypeStruct((N, Dc), F32)

    scratch = [
        pltpu.VMEM((2, K), jnp.int32),
        pltpu.VMEM((2, K), jnp.int32),
        pltpu.VMEM((K, Dc), F32),
        pltpu.VMEM((K, Dc), F32),
        pltpu.VMEM_SHARED((N, Dc), F32),
        pltpu.SemaphoreType.DMA,
        pltpu.SemaphoreType.DMA,
        pltpu.SemaphoreType.DMA,
        pltpu.SemaphoreType.DMA,
    ]

    def run(g_hbm, eidx_hbm, out_hbm, s, sidx0, sidx1, gbuf0, gbuf1, acc,
            gsem0, gsem1, isem0, isem1):
        @pl.when(s < 15)
        def _():
            pltpu.sync_copy(g_hbm.at[pl.ds(s * 624, 624)],
                            acc.at[pl.ds(s * 624, 624)])

        @pl.when(s == 15)
        def _():
            pltpu.sync_copy(g_hbm.at[pl.ds(9360, 640)],
                            acc.at[pl.ds(9360, 640)])

        plsc.subcore_barrier()

        pltpu.sync_copy(eidx_hbm.at[s].at[0], sidx0)
        pltpu.async_copy(g_hbm.at[sidx0.at[0]], gbuf0, gsem0)
        pltpu.async_copy(eidx_hbm.at[s].at[1], sidx1, isem1)

        def group(i, _):
            nxt = i + 1
            nnxt = i + 2

            @pl.when(jnp.logical_and(nxt < G, nxt % 2 == 1))
            def _():
                pltpu.make_async_copy(eidx_hbm.at[s].at[nxt], sidx1,
                                      isem1).wait()
                pltpu.async_copy(g_hbm.at[sidx1.at[0]], gbuf1, gsem1)

            @pl.when(jnp.logical_and(nxt < G, nxt % 2 == 0))
            def _():
                pltpu.make_async_copy(eidx_hbm.at[s].at[nxt], sidx0,
                                      isem0).wait()
                pltpu.async_copy(g_hbm.at[sidx0.at[0]], gbuf0, gsem0)

            @pl.when(i % 2 == 0)
            def _():
                pltpu.make_async_copy(g_hbm.at[sidx0.at[0]], gbuf0,
                                      gsem0).wait()
                pltpu.sync_copy(gbuf0, acc.at[sidx0.at[1]], add=True)

            @pl.when(i % 2 == 1)
            def _():
                pltpu.make_async_copy(g_hbm.at[sidx1.at[0]], gbuf1,
                                      gsem1).wait()
                pltpu.sync_copy(gbuf1, acc.at[sidx1.at[1]], add=True)

            @pl.when(jnp.logical_and(nnxt < G, nnxt % 2 == 0))
            def _():
                pltpu.async_copy(eidx_hbm.at[s].at[nnxt], sidx0, isem0)

            @pl.when(jnp.logical_and(nnxt < G, nnxt % 2 == 1))
            def _():
                pltpu.async_copy(eidx_hbm.at[s].at[nnxt], sidx1, isem1)

            return 0

        lax.fori_loop(0, G, group, 0)
        plsc.subcore_barrier()

        @pl.when(s < 15)
        def _():
            pltpu.sync_copy(acc.at[pl.ds(s * 624, 624)],
                            out_hbm.at[pl.ds(s * 624, 624)])

        @pl.when(s == 15)
        def _():
            pltpu.sync_copy(acc.at[pl.ds(9360, 640)],
                            out_hbm.at[pl.ds(9360, 640)])

    if two:
        @functools.partial(
            pl.kernel, out_type=[chunk_t, chunk_t], mesh=mesh,
            scratch_types=scratch)
        def prop(ga, gb, eidx_hbm, outa, outb,
                 sidx0, sidx1, gbuf0, gbuf1, acc, gsem0, gsem1,
                 isem0, isem1):
            c = lax.axis_index("c")
            s = lax.axis_index("s")

            @pl.when(c == 0)
            def _():
                run(ga, eidx_hbm, outa, s, sidx0, sidx1, gbuf0, gbuf1, acc,
                    gsem0, gsem1, isem0, isem1)

            @pl.when(c == 1)
            def _():
                run(gb, eidx_hbm, outb, s, sidx0, sidx1, gbuf0, gbuf1, acc,
                    gsem0, gsem1, isem0, isem1)
    else:
        @functools.partial(
            pl.kernel, out_type=chunk_t, mesh=mesh, scratch_types=scratch)
        def prop(ga, eidx_hbm, outa,
                 sidx0, sidx1, gbuf0, gbuf1, acc, gsem0, gsem1,
                 isem0, isem1):
            c = lax.axis_index("c")
            s = lax.axis_index("s")

            @pl.when(c == 0)
            def _():
                run(ga, eidx_hbm, outa, s, sidx0, sidx1, gbuf0, gbuf1, acc,
                    gsem0, gsem1, isem0, isem1)

    return prop


_prop2 = _make_prop(128, two=True)


def _make_deg():
    mesh = plsc.VectorSubcoreMesh(core_axis_name="c", subcore_axis_name="s")
    out_t = jax.ShapeDtypeStruct((N, 16), F32)
    scratch = [
        pltpu.VMEM((GD, KD), jnp.int32),
        pltpu.VMEM((KD, 16), F32),
        pltpu.VMEM_SHARED((N, 16), F32),
        pltpu.SemaphoreType.DMA,
    ]

    @functools.partial(pl.kernel, out_type=[out_t, out_t], mesh=mesh,
                       scratch_types=scratch)
    def deg(ones_hbm, dst_hbm, out0, out1, edst, ones, acc, ssem):
        c = lax.axis_index("c")
        s = lax.axis_index("s")
        pltpu.sync_copy(dst_hbm.at[c].at[s], edst)
        pltpu.sync_copy(ones_hbm.at[pl.ds(0, KD)], ones)

        @pl.when(s < 15)
        def _():
            pltpu.sync_copy(ones_hbm.at[pl.ds(s * 624, 624)],
                            acc.at[pl.ds(s * 624, 624)])

        @pl.when(s == 15)
        def _():
            pltpu.sync_copy(ones_hbm.at[pl.ds(9360, 640)],
                            acc.at[pl.ds(9360, 640)])

        plsc.subcore_barrier()

        def chunk(b, _):
            for j in range(5):
                pltpu.async_copy(ones, acc.at[edst.at[b * 5 + j]], ssem,
                                 add=True)
            for j in range(5):
                pltpu.make_async_copy(ones, acc.at[edst.at[b * 5 + j]],
                                      ssem).wait()
            return 0

        lax.fori_loop(0, GD // 5, chunk, 0)
        plsc.subcore_barrier()

        @pl.when(jnp.logical_and(c == 0, s < 15))
        def _():
            pltpu.sync_copy(acc.at[pl.ds(s * 624, 624)],
                            out0.at[pl.ds(s * 624, 624)])

        @pl.when(jnp.logical_and(c == 0, s == 15))
        def _():
            pltpu.sync_copy(acc.at[pl.ds(9360, 640)],
                            out0.at[pl.ds(9360, 640)])

        @pl.when(jnp.logical_and(c == 1, s < 15))
        def _():
            pltpu.sync_copy(acc.at[pl.ds(s * 624, 624)],
                            out1.at[pl.ds(s * 624, 624)])

        @pl.when(jnp.logical_and(c == 1, s == 15))
        def _():
            pltpu.sync_copy(acc.at[pl.ds(9360, 640)],
                            out1.at[pl.ds(9360, 640)])

    return deg


_deg = _make_deg()


BMF = 200
BM = 400


def _dinv(d0_ref, d1_ref):
    return lax.rsqrt(d0_ref[...][:, :1] + d1_ref[...][:, :1] - 1.0)


def _feat_body(x_ref, d0_ref, d1_ref, wf1, bf1, wf2, bf2, wf3, bf3, g0_ref):
    xb = x_ref[...]
    f2 = jnp.maximum(jnp.dot(xb[:, :21], wf2[...],
                             preferred_element_type=F32) + bf2[...], 0.0)
    f1 = jnp.maximum(jnp.dot(xb[:, 21:6165], wf1[...],
                             preferred_element_type=F32) + bf1[...], 0.0)
    f3 = jnp.maximum(jnp.dot(xb[:, 6165:], wf3[...],
                             preferred_element_type=F32) + bf3[...], 0.0)
    feat = jnp.concatenate([f2, f1, f3, jnp.zeros((BMF, 43), F32)], axis=1)
    g0_ref[...] = feat * _dinv(d0_ref, d1_ref)


def _feat(x, d0, d1, wf1, bf1, wf2, bf2, wf3, bf3):
    full = lambda r, c: pl.BlockSpec((r, c), lambda i: (0, 0))
    return pl.pallas_call(
        _feat_body,
        grid=(N // BMF,),
        in_specs=[
            pl.BlockSpec((BMF, 6485), lambda i: (i, 0)),
            pl.BlockSpec((BMF, 16), lambda i: (i, 0)),
            pl.BlockSpec((BMF, 16), lambda i: (i, 0)),
            full(6144, 128), full(1, 128),
            full(21, 21), full(1, 21),
            full(320, 320), full(1, 320),
        ],
        out_specs=pl.BlockSpec((BMF, 512), lambda i: (i, 0)),
        out_shape=jax.ShapeDtypeStruct((N, 512), F32),
    )(x, d0, d1, wf1, bf1, wf2, bf2, wf3, bf3)


def _mid1_body(s0_ref, d0_ref, d1_ref, wp1, bp1, wa1, ba1, g1_ref):
    dinv = _dinv(d0_ref, d1_ref)
    pf = s0_ref[...][:, :469] * dinv
    xh = jnp.maximum(jnp.dot(pf, wp1[...], preferred_element_type=F32)
                     + bp1[...], 0.0)
    yh = jnp.maximum(jnp.dot(pf, wa1[...], preferred_element_type=F32)
                     + ba1[...], 0.0)
    g1 = jnp.concatenate([xh, yh, jnp.zeros((BM, 86), F32)], axis=1)
    g1_ref[...] = g1 * dinv


def _mid1(s0, d0, d1, wp1, bp1, wa1, ba1):
    full = lambda r, c: pl.BlockSpec((r, c), lambda i: (0, 0))
    return pl.pallas_call(
        _mid1_body,
        grid=(N // BM,),
        in_specs=[
            pl.BlockSpec((BM, 512), lambda i: (i, 0)),
            pl.BlockSpec((BM, 16), lambda i: (i, 0)),
            pl.BlockSpec((BM, 16), lambda i: (i, 0)),
            full(469, 469), full(1, 469),
            full(469, 469), full(1, 469),
        ],
        out_specs=pl.BlockSpec((BM, 1024), lambda i: (i, 0)),
        out_shape=jax.ShapeDtypeStruct((N, 1024), F32),
    )(s0, d0, d1, wp1, bp1, wa1, ba1)


def _mid2_body(s1_ref, d0_ref, d1_ref, wp2, bp2, wa2, ba2, g2_ref):
    dinv = _dinv(d0_ref, d1_ref)
    s1 = s1_ref[...]
    tx = s1[:, :469] * dinv
    ty = s1[:, 469:938] * dinv
    xh = jnp.maximum(jnp.dot(tx, wp2[...], preferred_element_type=F32)
                     + bp2[...], 0.0)
    yh = jnp.maximum(jnp.dot(ty, wa2[...], preferred_element_type=F32)
                     + ba2[...], 0.0)
    g2 = jnp.concatenate([xh, yh, jnp.zeros((BM, 172), F32)], axis=1)
    g2_ref[...] = g2 * dinv


def _mid2(s1, d0, d1, wp2, bp2, wa2, ba2):
    full = lambda r, c: pl.BlockSpec((r, c), lambda i: (0, 0))
    return pl.pallas_call(
        _mid2_body,
        grid=(N // BM,),
        in_specs=[
            pl.BlockSpec((BM, 1024), lambda i: (i, 0)),
            pl.BlockSpec((BM, 16), lambda i: (i, 0)),
            pl.BlockSpec((BM, 16), lambda i: (i, 0)),
            full(469, 938), full(1, 938),
            full(469, 938), full(1, 938),
        ],
        out_specs=pl.BlockSpec((BM, 2048), lambda i: (i, 0)),
        out_shape=jax.ShapeDtypeStruct((N, 2048), F32),
    )(s1, d0, d1, wp2, bp2, wa2, ba2)


def _final_body(s2_ref, d0_ref, d1_ref, batch_ref, wp3, bp3, wg1, bg1, gam,
                bet, wg2, bg2, out_ref, sums, cnts):
    i = pl.program_id(0)
    nblk = pl.num_programs(0)

    @pl.when(i == 0)
    def _():
        sums[...] = jnp.zeros_like(sums)
        cnts[...] = jnp.zeros_like(cnts)

    u = s2_ref[...][:, :1876] * _dinv(d0_ref, d1_ref)
    z = jnp.maximum(jnp.dot(u, wp3[...], preferred_element_type=F32)
                    + bp3[...], 0.0)
    seg = batch_ref[0]
    oh = (lax.broadcasted_iota(jnp.int32, (32, BM), 0) == seg).astype(F32)
    sums[...] += jnp.dot(oh, z, preferred_element_type=F32)
    cnts[...] += jnp.sum(oh, axis=1, keepdims=True)

    @pl.when(i == nblk - 1)
    def _():
        pooled = sums[...] / jnp.maximum(cnts[...], 1.0)
        h = jnp.dot(pooled, wg1[...], preferred_element_type=F32) + bg1[...]
        mu = jnp.mean(h, axis=0, keepdims=True)
        var = jnp.mean((h - mu) ** 2, axis=0, keepdims=True)
        h = (h - mu) * lax.rsqrt(var + 1e-5) * gam[...] + bet[...]
        h = jnp.maximum(h, 0.0)
        o = jnp.dot(h, wg2[...], preferred_element_type=F32) + bg2[...]
        out_ref[...] = jax.nn.sigmoid(o)


def _final(s2, d0, d1, batch3d, wp3, bp3, wg1, bg1, gam, bet, wg2, bg2):
    full = lambda r, c: pl.BlockSpec((r, c), lambda i: (0, 0))
    return pl.pallas_call(
        _final_body,
        grid=(N // BM,),
        in_specs=[
            pl.BlockSpec((BM, 2048), lambda i: (i, 0)),
            pl.BlockSpec((BM, 16), lambda i: (i, 0)),
            pl.BlockSpec((BM, 16), lambda i: (i, 0)),
            pl.BlockSpec((1, 1, BM), lambda i: (i, 0, 0)),
            full(1876, 1876), full(1, 1876),
            full(1876, 1024), full(1, 1024),
            full(1, 1024), full(1, 1024),
            full(1024, 486), full(1, 486),
        ],
        out_specs=pl.BlockSpec((32, 486), lambda i: (0, 0)),
        out_shape=jax.ShapeDtypeStruct((32, 486), F32),
        scratch_shapes=[
            pltpu.VMEM((32, 1876), F32),
            pltpu.VMEM((32, 1), F32),
        ],
    )(s2, d0, d1, batch3d, wp3, bp3, wg1, bg1, gam, bet, wg2, bg2)


def kernel(x, edge_index, batch, W_f1, b_f1, W_f2, b_f2, W_f3, b_f3,
           W_p1, b_p1, W_p2, b_p2, W_a1, b_a1, W_a2, b_a2, W_p3, b_p3,
           W_g1, b_g1, gamma, beta, W_g2, b_g2):
    eidx = jnp.stack([edge_index[0].reshape(NS, G, K),
                      edge_index[1].reshape(NS, G, K)], axis=2)
    dst4 = edge_index[1].reshape(2, NS, GD, KD)
    ones = jnp.ones((N, 16), F32)
    row = lambda v: v.reshape(1, -1)

    d0, d1 = _deg(ones, dst4)

    def prop(g, width):
        half = width // 2
        nc = half // 128
        parts = [_prop2(g[:, i * 128:(i + 1) * 128],
                        g[:, half + i * 128:half + (i + 1) * 128],
                        eidx)
                 for i in range(nc)]
        return jnp.concatenate([ab[0] for ab in parts]
                               + [ab[1] for ab in parts], axis=1)

    g0 = _feat(x, d0, d1, W_f1, row(b_f1), W_f2, row(b_f2), W_f3, row(b_f3))
    s0 = prop(g0, 512)
    g1 = _mid1(s0, d0, d1, W_p1, row(b_p1), W_a1, row(b_a1))
    s1 = prop(g1, 1024)
    g2 = _mid2(s1, d0, d1, W_p2, row(b_p2), W_a2, row(b_a2))
    s2 = prop(g2, 2048)

    out = _final(s2, d0, d1, batch.reshape(N // BM, 1, BM), W_p3, row(b_p3),
                 W_g1, row(b_g1), row(gamma), row(beta), W_g2, row(b_g2))
    return out


# speedup vs baseline: 7.8442x; 1.1329x over previous
import functools

import jax
import jax.numpy as jnp
from jax import lax
from jax.experimental import pallas as pl
from jax.experimental.pallas import tpu as pltpu
from jax.experimental.pallas import tpu_sc as plsc

N = 10000
E = 160000
NS = 16
ES = E // NS
K = 80
G = ES // K
ESD = E // (2 * NS)
KD = 40
GD = ESD // KD
F32 = jnp.float32


def _make_prop(Dc, two):
    assert Dc % 128 == 0 and N * Dc * 4 <= 8 * 1024 * 1024
    mesh = plsc.VectorSubcoreMesh(core_axis_name="c", subcore_axis_name="s")
    chunk_t = jax.ShapeDtypeStruct((N, Dc), F32)

    scratch = [
        pltpu.VMEM((2, K), jnp.int32),
        pltpu.VMEM((2, K), jnp.int32),
        pltpu.VMEM((K,), jnp.int32),
        pltpu.VMEM((K,), jnp.int32),
        pltpu.VMEM((K, Dc), F32),
        pltpu.VMEM((K, Dc), F32),
        pltpu.VMEM_SHARED((N, Dc), F32),
        pltpu.SemaphoreType.DMA,
        pltpu.SemaphoreType.DMA,
        pltpu.SemaphoreType.DMA,
        pltpu.SemaphoreType.DMA,
        pltpu.SemaphoreType.DMA,
        pltpu.SemaphoreType.DMA,
    ]

    def run(g_hbm, eidx_hbm, out_hbm, s, sidx0, sidx1, sdst0, sdst1,
            gbuf0, gbuf1, acc, gsem0, gsem1, isem0, isem1, ssem0, ssem1):
        @pl.when(s < 15)
        def _():
            pltpu.sync_copy(g_hbm.at[pl.ds(s * 624, 624)],
                            acc.at[pl.ds(s * 624, 624)])

        @pl.when(s == 15)
        def _():
            pltpu.sync_copy(g_hbm.at[pl.ds(9360, 640)],
                            acc.at[pl.ds(9360, 640)])

        plsc.subcore_barrier()

        pltpu.sync_copy(eidx_hbm.at[s].at[0], sidx0)
        pltpu.async_copy(g_hbm.at[sidx0.at[0]], gbuf0, gsem0)
        pltpu.async_copy(eidx_hbm.at[s].at[1], sidx1, isem1)

        def group(i, _):
            nxt = i + 1
            nnxt = i + 2

            @pl.when(jnp.logical_and(nxt < G, nxt % 2 == 1))
            def _():
                pltpu.make_async_copy(eidx_hbm.at[s].at[nxt], sidx1,
                                      isem1).wait()

                @pl.when(i > 0)
                def _():
                    pltpu.make_async_copy(gbuf1, acc.at[sdst1], ssem1).wait()

                pltpu.async_copy(g_hbm.at[sidx1.at[0]], gbuf1, gsem1)

            @pl.when(jnp.logical_and(nxt < G, nxt % 2 == 0))
            def _():
                pltpu.make_async_copy(eidx_hbm.at[s].at[nxt], sidx0,
                                      isem0).wait()
                pltpu.make_async_copy(gbuf0, acc.at[sdst0], ssem0).wait()
                pltpu.async_copy(g_hbm.at[sidx0.at[0]], gbuf0, gsem0)

            @pl.when(i % 2 == 0)
            def _():
                pltpu.make_async_copy(g_hbm.at[sidx0.at[0]], gbuf0,
                                      gsem0).wait()
                for j in range(K // 16):
                    sdst0[pl.ds(j * 16, 16)] = sidx0[1, pl.ds(j * 16, 16)]
                pltpu.async_copy(gbuf0, acc.at[sdst0], ssem0, add=True)

            @pl.when(i % 2 == 1)
            def _():
                pltpu.make_async_copy(g_hbm.at[sidx1.at[0]], gbuf1,
                                      gsem1).wait()
                for j in range(K // 16):
                    sdst1[pl.ds(j * 16, 16)] = sidx1[1, pl.ds(j * 16, 16)]
                pltpu.async_copy(gbuf1, acc.at[sdst1], ssem1, add=True)

            @pl.when(jnp.logical_and(nnxt < G, nnxt % 2 == 0))
            def _():
                pltpu.async_copy(eidx_hbm.at[s].at[nnxt], sidx0, isem0)

            @pl.when(jnp.logical_and(nnxt < G, nnxt % 2 == 1))
            def _():
                pltpu.async_copy(eidx_hbm.at[s].at[nnxt], sidx1, isem1)

            return 0

        lax.fori_loop(0, G, group, 0)
        pltpu.make_async_copy(gbuf0, acc.at[sdst0], ssem0).wait()
        pltpu.make_async_copy(gbuf1, acc.at[sdst1], ssem1).wait()
        plsc.subcore_barrier()

        @pl.when(s < 15)
        def _():
            pltpu.sync_copy(acc.at[pl.ds(s * 624, 624)],
                            out_hbm.at[pl.ds(s * 624, 624)])

        @pl.when(s == 15)
        def _():
            pltpu.sync_copy(acc.at[pl.ds(9360, 640)],
                            out_hbm.at[pl.ds(9360, 640)])

    if two:
        @functools.partial(
            pl.kernel, out_type=[chunk_t, chunk_t], mesh=mesh,
            scratch_types=scratch)
        def prop(ga, gb, eidx_hbm, outa, outb,
                 sidx0, sidx1, sdst0, sdst1, gbuf0, gbuf1, acc,
                 gsem0, gsem1, isem0, isem1, ssem0, ssem1):
            c = lax.axis_index("c")
            s = lax.axis_index("s")

            @pl.when(c == 0)
            def _():
                run(ga, eidx_hbm, outa, s, sidx0, sidx1, sdst0, sdst1,
                    gbuf0, gbuf1, acc, gsem0, gsem1, isem0, isem1,
                    ssem0, ssem1)

            @pl.when(c == 1)
            def _():
                run(gb, eidx_hbm, outb, s, sidx0, sidx1, sdst0, sdst1,
                    gbuf0, gbuf1, acc, gsem0, gsem1, isem0, isem1,
                    ssem0, ssem1)
    else:
        @functools.partial(
            pl.kernel, out_type=chunk_t, mesh=mesh, scratch_types=scratch)
        def prop(ga, eidx_hbm, outa,
                 sidx0, sidx1, sdst0, sdst1, gbuf0, gbuf1, acc,
                 gsem0, gsem1, isem0, isem1, ssem0, ssem1):
            c = lax.axis_index("c")
            s = lax.axis_index("s")

            @pl.when(c == 0)
            def _():
                run(ga, eidx_hbm, outa, s, sidx0, sidx1, sdst0, sdst1,
                    gbuf0, gbuf1, acc, gsem0, gsem1, isem0, isem1,
                    ssem0, ssem1)

    return prop


_prop2 = _make_prop(128, two=True)


def _make_deg():
    mesh = plsc.VectorSubcoreMesh(core_axis_name="c", subcore_axis_name="s")
    out_t = jax.ShapeDtypeStruct((N, 16), F32)
    scratch = [
        pltpu.VMEM((GD, KD), jnp.int32),
        pltpu.VMEM((KD, 16), F32),
        pltpu.VMEM_SHARED((N, 16), F32),
        pltpu.SemaphoreType.DMA,
    ]

    @functools.partial(pl.kernel, out_type=[out_t, out_t], mesh=mesh,
                       scratch_types=scratch)
    def deg(ones_hbm, dst_hbm, out0, out1, edst, ones, acc, ssem):
        c = lax.axis_index("c")
        s = lax.axis_index("s")
        pltpu.sync_copy(dst_hbm.at[c].at[s], edst)
        pltpu.sync_copy(ones_hbm.at[pl.ds(0, KD)], ones)

        @pl.when(s < 15)
        def _():
            pltpu.sync_copy(ones_hbm.at[pl.ds(s * 624, 624)],
                            acc.at[pl.ds(s * 624, 624)])

        @pl.when(s == 15)
        def _():
            pltpu.sync_copy(ones_hbm.at[pl.ds(9360, 640)],
                            acc.at[pl.ds(9360, 640)])

        plsc.subcore_barrier()

        def chunk(b, _):
            for j in range(5):
                pltpu.async_copy(ones, acc.at[edst.at[b * 5 + j]], ssem,
                                 add=True)
            for j in range(5):
                pltpu.make_async_copy(ones, acc.at[edst.at[b * 5 + j]],
                                      ssem).wait()
            return 0

        lax.fori_loop(0, GD // 5, chunk, 0)
        plsc.subcore_barrier()

        @pl.when(jnp.logical_and(c == 0, s < 15))
        def _():
            pltpu.sync_copy(acc.at[pl.ds(s * 624, 624)],
                            out0.at[pl.ds(s * 624, 624)])

        @pl.when(jnp.logical_and(c == 0, s == 15))
        def _():
            pltpu.sync_copy(acc.at[pl.ds(9360, 640)],
                            out0.at[pl.ds(9360, 640)])

        @pl.when(jnp.logical_and(c == 1, s < 15))
        def _():
            pltpu.sync_copy(acc.at[pl.ds(s * 624, 624)],
                            out1.at[pl.ds(s * 624, 624)])

        @pl.when(jnp.logical_and(c == 1, s == 15))
        def _():
            pltpu.sync_copy(acc.at[pl.ds(9360, 640)],
                            out1.at[pl.ds(9360, 640)])

    return deg


_deg = _make_deg()


BMF = 200
BM = 400


def _dinv(d0_ref, d1_ref):
    return lax.rsqrt(d0_ref[...][:, :1] + d1_ref[...][:, :1] - 1.0)


def _feat_body(x_ref, d0_ref, d1_ref, wf1, bf1, wf2, bf2, wf3, bf3, g0_ref):
    xb = x_ref[...]
    f2 = jnp.maximum(jnp.dot(xb[:, :21], wf2[...],
                             preferred_element_type=F32) + bf2[...], 0.0)
    f1 = jnp.maximum(jnp.dot(xb[:, 21:6165], wf1[...],
                             preferred_element_type=F32) + bf1[...], 0.0)
    f3 = jnp.maximum(jnp.dot(xb[:, 6165:], wf3[...],
                             preferred_element_type=F32) + bf3[...], 0.0)
    feat = jnp.concatenate([f2, f1, f3, jnp.zeros((BMF, 43), F32)], axis=1)
    g0_ref[...] = feat * _dinv(d0_ref, d1_ref)


def _feat(x, d0, d1, wf1, bf1, wf2, bf2, wf3, bf3):
    full = lambda r, c: pl.BlockSpec((r, c), lambda i: (0, 0))
    return pl.pallas_call(
        _feat_body,
        grid=(N // BMF,),
        in_specs=[
            pl.BlockSpec((BMF, 6485), lambda i: (i, 0)),
            pl.BlockSpec((BMF, 16), lambda i: (i, 0)),
            pl.BlockSpec((BMF, 16), lambda i: (i, 0)),
            full(6144, 128), full(1, 128),
            full(21, 21), full(1, 21),
            full(320, 320), full(1, 320),
        ],
        out_specs=pl.BlockSpec((BMF, 512), lambda i: (i, 0)),
        out_shape=jax.ShapeDtypeStruct((N, 512), F32),
    )(x, d0, d1, wf1, bf1, wf2, bf2, wf3, bf3)


def _mid1_body(s0_ref, d0_ref, d1_ref, wp1, bp1, wa1, ba1, g1_ref):
    dinv = _dinv(d0_ref, d1_ref)
    pf = s0_ref[...][:, :469] * dinv
    xh = jnp.maximum(jnp.dot(pf, wp1[...], preferred_element_type=F32)
                     + bp1[...], 0.0)
    yh = jnp.maximum(jnp.dot(pf, wa1[...], preferred_element_type=F32)
                     + ba1[...], 0.0)
    g1 = jnp.concatenate([xh, yh, jnp.zeros((BM, 86), F32)], axis=1)
    g1_ref[...] = g1 * dinv


def _mid1(s0, d0, d1, wp1, bp1, wa1, ba1):
    full = lambda r, c: pl.BlockSpec((r, c), lambda i: (0, 0))
    return pl.pallas_call(
        _mid1_body,
        grid=(N // BM,),
        in_specs=[
            pl.BlockSpec((BM, 512), lambda i: (i, 0)),
            pl.BlockSpec((BM, 16), lambda i: (i, 0)),
            pl.BlockSpec((BM, 16), lambda i: (i, 0)),
            full(469, 469), full(1, 469),
            full(469, 469), full(1, 469),
        ],
        out_specs=pl.BlockSpec((BM, 1024), lambda i: (i, 0)),
        out_shape=jax.ShapeDtypeStruct((N, 1024), F32),
    )(s0, d0, d1, wp1, bp1, wa1, ba1)


def _mid2_body(s1_ref, d0_ref, d1_ref, wp2, bp2, wa2, ba2, g2_ref):
    dinv = _dinv(d0_ref, d1_ref)
    s1 = s1_ref[...]
    tx = s1[:, :469] * dinv
    ty = s1[:, 469:938] * dinv
    xh = jnp.maximum(jnp.dot(tx, wp2[...], preferred_element_type=F32)
                     + bp2[...], 0.0)
    yh = jnp.maximum(jnp.dot(ty, wa2[...], preferred_element_type=F32)
                     + ba2[...], 0.0)
    g2 = jnp.concatenate([xh, yh, jnp.zeros((BM, 172), F32)], axis=1)
    g2_ref[...] = g2 * dinv


def _mid2(s1, d0, d1, wp2, bp2, wa2, ba2):
    full = lambda r, c: pl.BlockSpec((r, c), lambda i: (0, 0))
    return pl.pallas_call(
        _mid2_body,
        grid=(N // BM,),
        in_specs=[
            pl.BlockSpec((BM, 1024), lambda i: (i, 0)),
            pl.BlockSpec((BM, 16), lambda i: (i, 0)),
            pl.BlockSpec((BM, 16), lambda i: (i, 0)),
            full(469, 938), full(1, 938),
            full(469, 938), full(1, 938),
        ],
        out_specs=pl.BlockSpec((BM, 2048), lambda i: (i, 0)),
        out_shape=jax.ShapeDtypeStruct((N, 2048), F32),
    )(s1, d0, d1, wp2, bp2, wa2, ba2)


def _final_body(s2_ref, d0_ref, d1_ref, batch_ref, wp3, bp3, wg1, bg1, gam,
                bet, wg2, bg2, out_ref, sums, cnts):
    i = pl.program_id(0)
    nblk = pl.num_programs(0)

    @pl.when(i == 0)
    def _():
        sums[...] = jnp.zeros_like(sums)
        cnts[...] = jnp.zeros_like(cnts)

    u = s2_ref[...][:, :1876] * _dinv(d0_ref, d1_ref)
    z = jnp.maximum(jnp.dot(u, wp3[...], preferred_element_type=F32)
                    + bp3[...], 0.0)
    seg = batch_ref[0]
    oh = (lax.broadcasted_iota(jnp.int32, (32, BM), 0) == seg).astype(F32)
    sums[...] += jnp.dot(oh, z, preferred_element_type=F32)
    cnts[...] += jnp.sum(oh, axis=1, keepdims=True)

    @pl.when(i == nblk - 1)
    def _():
        pooled = sums[...] / jnp.maximum(cnts[...], 1.0)
        h = jnp.dot(pooled, wg1[...], preferred_element_type=F32) + bg1[...]
        mu = jnp.mean(h, axis=0, keepdims=True)
        var = jnp.mean((h - mu) ** 2, axis=0, keepdims=True)
        h = (h - mu) * lax.rsqrt(var + 1e-5) * gam[...] + bet[...]
        h = jnp.maximum(h, 0.0)
        o = jnp.dot(h, wg2[...], preferred_element_type=F32) + bg2[...]
        out_ref[...] = jax.nn.sigmoid(o)


def _final(s2, d0, d1, batch3d, wp3, bp3, wg1, bg1, gam, bet, wg2, bg2):
    full = lambda r, c: pl.BlockSpec((r, c), lambda i: (0, 0))
    return pl.pallas_call(
        _final_body,
        grid=(N // BM,),
        in_specs=[
            pl.BlockSpec((BM, 2048), lambda i: (i, 0)),
            pl.BlockSpec((BM, 16), lambda i: (i, 0)),
            pl.BlockSpec((BM, 16), lambda i: (i, 0)),
            pl.BlockSpec((1, 1, BM), lambda i: (i, 0, 0)),
            full(1876, 1876), full(1, 1876),
            full(1876, 1024), full(1, 1024),
            full(1, 1024), full(1, 1024),
            full(1024, 486), full(1, 486),
        ],
        out_specs=pl.BlockSpec((32, 486), lambda i: (0, 0)),
        out_shape=jax.ShapeDtypeStruct((32, 486), F32),
        scratch_shapes=[
            pltpu.VMEM((32, 1876), F32),
            pltpu.VMEM((32, 1), F32),
        ],
    )(s2, d0, d1, batch3d, wp3, bp3, wg1, bg1, gam, bet, wg2, bg2)


def kernel(x, edge_index, batch, W_f1, b_f1, W_f2, b_f2, W_f3, b_f3,
           W_p1, b_p1, W_p2, b_p2, W_a1, b_a1, W_a2, b_a2, W_p3, b_p3,
           W_g1, b_g1, gamma, beta, W_g2, b_g2):
    eidx = jnp.stack([edge_index[0].reshape(NS, G, K),
                      edge_index[1].reshape(NS, G, K)], axis=2)
    dst4 = edge_index[1].reshape(2, NS, GD, KD)
    ones = jnp.ones((N, 16), F32)
    row = lambda v: v.reshape(1, -1)

    d0, d1 = _deg(ones, dst4)

    def prop(g, width):
        half = width // 2
        nc = half // 128
        parts = [_prop2(g[:, i * 128:(i + 1) * 128],
                        g[:, half + i * 128:half + (i + 1) * 128],
                        eidx)
                 for i in range(nc)]
        return jnp.concatenate([ab[0] for ab in parts]
                               + [ab[1] for ab in parts], axis=1)

    g0 = _feat(x, d0, d1, W_f1, row(b_f1), W_f2, row(b_f2), W_f3, row(b_f3))
    s0 = prop(g0, 512)
    g1 = _mid1(s0, d0, d1, W_p1, row(b_p1), W_a1, row(b_a1))
    s1 = prop(g1, 1024)
    g2 = _mid2(s1, d0, d1, W_p2, row(b_p2), W_a2, row(b_a2))
    s2 = prop(g2, 2048)

    out = _final(s2, d0, d1, batch.reshape(N // BM, 1, BM), W_p3, row(b_p3),
                 W_g1, row(b_g1), row(gamma), row(beta), W_g2, row(b_g2))
    return out
